# baseline (speedup 1.0000x reference)
import functools

import jax
import jax.numpy as jnp
from jax import lax
from jax.experimental import pallas as pl
from jax.experimental.pallas import tpu as pltpu
from jax.experimental.pallas import tpu_sc as plsc

_NUM_EMB = 100000
_NUM_NEW = 1024
_FEATURES = 128
_BATCH = 16384
_HIST = 50
_HPAD = 56

_NC, _NS = 2, 16
_NW = _NC * _NS
_BT_PER_W = _BATCH // _NW
_KB = 4
_GRP = _BT_PER_W // _KB
_ROWS = _KB * _HIST
_GW = _KB * _HPAD
_NVR = _GW // 16
_NSLOT = 4

_NBUF = 2


def _body(xc_hbm, xo_hbm, emb_hbm, new_hbm, out_hbm, lst_v, org_v, bufs,
          fixs, fixidx, gsems, osems, fsems):
    c = lax.axis_index("c")
    s = lax.axis_index("s")
    wid = s * _NC + c
    bt0 = wid * _BT_PER_W
    iota16 = lax.iota(jnp.int32, 16)
    one16 = iota16 * 0 + 1
    zero16 = iota16 * 0

    def popcnt(m):
        return jnp.sum(jnp.where(m, one16, zero16))

    pltpu.sync_copy(xc_hbm.at[pl.ds(bt0 * _HPAD, _BT_PER_W * _HPAD)], lst_v)
    pltpu.sync_copy(xo_hbm.at[pl.ds(bt0 * _HPAD, _BT_PER_W * _HPAD)], org_v)

    def wait_gather(b):
        pltpu.make_async_copy(
            emb_hbm.at[pl.ds(0, _ROWS)], bufs[b], gsems[b]).wait()

    def wait_scatter(b):
        for kb in range(_KB):
            pltpu.make_async_copy(bufs[b].at[pl.ds(kb * _HIST, _HIST)],
                                  out_hbm.at[bt0], osems[b]).wait()

    def fire_gather(g, b):
        for kb in range(_KB):
            i = g * _KB + kb
            pltpu.async_copy(
                emb_hbm.at[lst_v.at[pl.ds(i * _HPAD, _HIST)]],
                bufs[b].at[pl.ds(kb * _HIST, _HIST)], gsems[b])

    def fire_scatter(g, b):
        for kb in range(_KB):
            pltpu.async_copy(bufs[b].at[pl.ds(kb * _HIST, _HIST)],
                             out_hbm.at[bt0 + g * _KB + kb], osems[b])

    def detect_fire_fix(g, b, fire_ok):
        q = jnp.int32(0)
        for u in range(_NVR):
            v = org_v[pl.ds(g * _GW + u * 16, 16)]
            m = v >= _NUM_EMB
            has = popcnt(m) > 0

            @pl.when(fire_ok & has & (q < _NSLOT))
            def _(v=v, m=m, q=q):
                idx2 = jnp.where(m, v - _NUM_EMB, iota16)
                fixidx[b][pl.ds(q * 16, 16)] = idx2
                pltpu.async_copy(new_hbm.at[fixidx[b].at[pl.ds(q * 16, 16)]],
                                 fixs[b].at[pl.ds(q * 16, 16)], fsems[b])

            q = q + jnp.where(has, 1, 0)
        return q

    def apply_rows(b, fixrow0, mm, rowvec):
        def acond(mc):
            return popcnt(mc) > 0

        def abody(mc):
            fl = plsc.all_reduce_ffs(mc)
            l_s = fl if fl.ndim == 0 else jnp.max(fl)
            row_s = jnp.max(jnp.where(iota16 == l_s, rowvec, 0))
            for cg in range(8):
                bufs[b][row_s, pl.ds(cg * 16, 16)] = (
                    fixs[b][fixrow0 + l_s, pl.ds(cg * 16, 16)])
            return mc & (iota16 != l_s)

        lax.while_loop(acond, abody, mm)

    def apply_fix(g, b, qf):
        def wbody(i, _):
            pltpu.make_async_copy(new_hbm.at[pl.ds(0, 16)],
                                  fixs[b].at[pl.ds(0, 16)], fsems[b]).wait()
            return 0

        lax.fori_loop(0, jnp.minimum(qf, _NSLOT), wbody, 0)

        q = jnp.int32(0)
        for u in range(_NVR):
            v = org_v[pl.ds(g * _GW + u * 16, 16)]
            m = v >= _NUM_EMB
            has = popcnt(m) > 0
            w = u * 16 + iota16
            rowvec = (w // _HPAD) * _HIST + w % _HPAD

            @pl.when(has & (q < _NSLOT))
            def _(m=m, q=q, rowvec=rowvec):
                apply_rows(b, q * 16, m, rowvec)

            @pl.when(has & (q >= _NSLOT))
            def _(v=v, m=m, rowvec=rowvec):
                idx2 = jnp.where(m, v - _NUM_EMB, iota16)
                fixidx[b][pl.ds(0, 16)] = idx2
                pltpu.async_copy(new_hbm.at[fixidx[b].at[pl.ds(0, 16)]],
                                 fixs[b].at[pl.ds(0, 16)], fsems[b])
                pltpu.make_async_copy(new_hbm.at[pl.ds(0, 16)],
                                      fixs[b].at[pl.ds(0, 16)],
                                      fsems[b]).wait()
                apply_rows(b, 0, m, rowvec)

            q = q + jnp.where(has, 1, 0)

    qs = []
    for b in range(_NBUF):
        qs.append(detect_fire_fix(b, b, jnp.bool_(True)))
        fire_gather(b, b)

    def body(gg, q):
        qn = []
        for b in range(_NBUF):
            g = gg * _NBUF + b
            wait_gather(b)
            apply_fix(g, b, q[b])
            fire_scatter(g, b)
            more = gg < _GRP // _NBUF - 1

            @pl.when(more)
            def _():
                wait_scatter(b)

            qnext = detect_fire_fix(g + _NBUF, b, more)

            @pl.when(more)
            def _(g=g, b=b):
                fire_gather(g + _NBUF, b)

            qn.append(qnext)
        return tuple(qn)

    lax.fori_loop(0, _GRP // _NBUF, body, tuple(qs))
    for b in range(_NBUF):
        wait_scatter(b)


def kernel(x, embedding, new_embedding):
    xi = x.astype(jnp.int32)
    xc = jnp.minimum(xi, _NUM_EMB - 1)
    pad = ((0, 0), (0, _HPAD - _HIST))
    xcp = jnp.pad(xc, pad).reshape(-1)
    xop = jnp.pad(xi, pad).reshape(-1)

    mesh = plsc.VectorSubcoreMesh(core_axis_name="c", subcore_axis_name="s",
                                  num_cores=_NC)
    run = pl.kernel(
        _body,
        out_type=jax.ShapeDtypeStruct((_BATCH, _HIST, _FEATURES),
                                      jnp.float32),
        mesh=mesh,
        compiler_params=pltpu.CompilerParams(needs_layout_passes=False),
        scratch_types=[
            pltpu.VMEM((_BT_PER_W * _HPAD,), jnp.int32),
            pltpu.VMEM((_BT_PER_W * _HPAD,), jnp.int32),
            tuple(pltpu.VMEM((_ROWS, _FEATURES), jnp.float32)
                  for _ in range(_NBUF)),
            tuple(pltpu.VMEM((_NSLOT * 16, _FEATURES), jnp.float32)
                  for _ in range(_NBUF)),
            tuple(pltpu.VMEM((_NSLOT * 16,), jnp.int32)
                  for _ in range(_NBUF)),
            tuple(pltpu.SemaphoreType.DMA for _ in range(_NBUF)),
            tuple(pltpu.SemaphoreType.DMA for _ in range(_NBUF)),
            tuple(pltpu.SemaphoreType.DMA for _ in range(_NBUF)),
        ],
    )
    return run(xcp, xop, embedding, new_embedding)


# speedup vs baseline: 1.8778x; 1.8778x over previous
import functools

import jax
import jax.numpy as jnp
from jax import lax
from jax.experimental import pallas as pl
from jax.experimental.pallas import tpu as pltpu
from jax.experimental.pallas import tpu_sc as plsc

_NUM_EMB = 100000
_NUM_NEW = 1024
_TABLE = _NUM_EMB + _NUM_NEW
_FEATURES = 128
_BATCH = 16384
_HIST = 50
_HPAD = 56

_NC, _NS = 2, 16
_NW = _NC * _NS
_BT_PER_W = _BATCH // _NW
_KB = 4
_GRP = _BT_PER_W // _KB
_ROWS = _KB * _HIST

_CP_CH = _NUM_EMB // _ROWS
_CP_SLOTS = 32
_NEW_CH = _NUM_NEW // _ROWS
_NEW_TAIL = _NUM_NEW - _NEW_CH * _ROWS

_NBUF = 3


def _gather_body(x_hbm, emb_hbm, new_hbm, out_hbm, table_s, idx_v, bufs,
                 gsems, osems):
    c = lax.axis_index("c")
    s = lax.axis_index("s")
    wid = s * _NC + c
    bt0 = wid * _BT_PER_W

    pltpu.sync_copy(x_hbm.at[pl.ds(bt0 * _HPAD, _BT_PER_W * _HPAD)], idx_v)

    def p0_wait_in(b):
        pltpu.make_async_copy(
            emb_hbm.at[pl.ds(0, _ROWS)], bufs[b], gsems[b]).wait()

    def p0_wait_out(b):
        pltpu.make_async_copy(
            bufs[b], table_s.at[c, pl.ds(0, _ROWS)], osems[b]).wait()

    def p0_fire_in(j, b):
        pltpu.async_copy(emb_hbm.at[pl.ds(j * _ROWS, _ROWS)], bufs[b],
                         gsems[b])

    def p0_fire_out(j, b):
        pltpu.async_copy(bufs[b], table_s.at[c, pl.ds(j * _ROWS, _ROWS)],
                         osems[b])

    for b in range(_NBUF):
        p0_fire_in(s + 16 * b, b)

    def p0_body(kk, _):
        for b in range(_NBUF):
            k = kk * _NBUF + b
            j = s + 16 * k
            actn = (k + _NBUF < _CP_SLOTS - 1) | ((k + _NBUF == _CP_SLOTS - 1)
                                                  & (s < 4))
            p0_wait_in(b)
            p0_fire_out(j, b)

            @pl.when(actn)
            def _():
                p0_wait_out(b)
                p0_fire_in(s + 16 * (k + _NBUF), b)
        return 0

    lax.fori_loop(0, (_CP_SLOTS - 2) // _NBUF, p0_body, 0)
    p0_wait_in(0)
    p0_fire_out(s + 16 * (_CP_SLOTS - 2), 0)

    @pl.when(s < 4)
    def _():
        p0_wait_in(1)
        p0_fire_out(s + 16 * (_CP_SLOTS - 1), 1)

    for b in range(_NBUF):
        p0_wait_out(b)

    @pl.when(s < _NEW_CH)
    def _():
        pltpu.sync_copy(new_hbm.at[pl.ds(s * _ROWS, _ROWS)], bufs[0])
        pltpu.sync_copy(bufs[0],
                        table_s.at[c, pl.ds(_NUM_EMB + s * _ROWS, _ROWS)])

    @pl.when(s == _NEW_CH)
    def _():
        pltpu.sync_copy(new_hbm.at[pl.ds(_NEW_CH * _ROWS, _NEW_TAIL)],
                        bufs[0].at[pl.ds(0, _NEW_TAIL)])
        pltpu.sync_copy(
            bufs[0].at[pl.ds(0, _NEW_TAIL)],
            table_s.at[c, pl.ds(_NUM_EMB + _NEW_CH * _ROWS, _NEW_TAIL)])

    plsc.subcore_barrier()

    def wait_gather(b):
        pltpu.make_async_copy(
            table_s.at[c, pl.ds(0, _ROWS)], bufs[b], gsems[b]).wait()

    def wait_scatter(b):
        pltpu.make_async_copy(
            bufs[b], table_s.at[c, pl.ds(0, _ROWS)], osems[b]).wait()

    def fire_gather(g, b):
        for kb in range(_KB):
            i = g * _KB + kb
            pltpu.async_copy(
                table_s.at[c].at[idx_v.at[pl.ds(i * _HPAD, _HIST)]],
                bufs[b].at[pl.ds(kb * _HIST, _HIST)], gsems[b])

    def fire_scatter(g, b):
        for kb in range(_KB):
            pltpu.async_copy(bufs[b].at[pl.ds(kb * _HIST, _HIST)],
                             out_hbm.at[bt0 + g * _KB + kb], osems[b])

    for b in range(_NBUF):
        fire_gather(b, b)

    def body(gg, _):
        for b in range(_NBUF):
            g = gg * _NBUF + b
            wait_gather(b)
            fire_scatter(g, b)

            @pl.when(g + _NBUF < _GRP)
            def _():
                wait_scatter(b)
                fire_gather(g + _NBUF, b)
        return 0

    lax.fori_loop(0, _GRP // _NBUF, body, 0)
    for g, b in ((_GRP - 2, 0), (_GRP - 1, 1)):
        wait_gather(b)
        fire_scatter(g, b)
    for b in range(_NBUF):
        wait_scatter(b)


def kernel(x, embedding, new_embedding):
    xi = x.astype(jnp.int32)
    xpad = jnp.pad(xi, ((0, 0), (0, _HPAD - _HIST))).reshape(-1)

    mesh = plsc.VectorSubcoreMesh(core_axis_name="c", subcore_axis_name="s")
    run = pl.kernel(
        _gather_body,
        out_type=jax.ShapeDtypeStruct((_BATCH, _HIST, _FEATURES),
                                      jnp.float32),
        mesh=mesh,
        scratch_types=[
            pltpu.HBM((_NC, _TABLE, _FEATURES), jnp.float32),
            pltpu.VMEM((_BT_PER_W * _HPAD,), jnp.int32),
            tuple(pltpu.VMEM((_ROWS, _FEATURES), jnp.float32)
                  for _ in range(_NBUF)),
            tuple(pltpu.SemaphoreType.DMA for _ in range(_NBUF)),
            tuple(pltpu.SemaphoreType.DMA for _ in range(_NBUF)),
        ],
    )
    return run(xpad, embedding, new_embedding)


# speedup vs baseline: 1.8798x; 1.0011x over previous
import jax
import jax.numpy as jnp
from jax import lax
from jax.experimental import pallas as pl
from jax.experimental.pallas import tpu as pltpu
from jax.experimental.pallas import tpu_sc as plsc

_NUM_EMB = 100000
_NUM_NEW = 1024
_TABLE = _NUM_EMB + _NUM_NEW
_FEATURES = 128
_BATCH = 16384
_HIST = 50
_HPAD = 56

_NC, _NS = 2, 16
_NW = _NC * _NS
_BT_PER_W = _BATCH // _NW
_KB = 4
_GRP = _BT_PER_W // _KB
_ROWS = _KB * _HIST

_CP_CH = _NUM_EMB // _ROWS
_CP_SLOTS = 32
_NEW_CH = _NUM_NEW // _ROWS
_NEW_TAIL = _NUM_NEW - _NEW_CH * _ROWS

_NBUF = 3


def _gather_body(x_hbm, emb_hbm, new_hbm, out_hbm, table_s, idx_v, bufs,
                 gsems, osems):
    c = lax.axis_index("c")
    s = lax.axis_index("s")
    wid = s * _NC + c
    bt0 = wid * _BT_PER_W

    pltpu.sync_copy(x_hbm.at[pl.ds(bt0 * _HPAD, _BT_PER_W * _HPAD)], idx_v)

    def p0_wait_in(b):
        pltpu.make_async_copy(
            emb_hbm.at[pl.ds(0, _ROWS)], bufs[b], gsems[b]).wait()

    def p0_wait_out(b):
        pltpu.make_async_copy(
            bufs[b], table_s.at[c, pl.ds(0, _ROWS)], osems[b]).wait()

    def p0_fire_in(j, b):
        pltpu.async_copy(emb_hbm.at[pl.ds(j * _ROWS, _ROWS)], bufs[b],
                         gsems[b])

    def p0_fire_out(j, b):
        pltpu.async_copy(bufs[b], table_s.at[c, pl.ds(j * _ROWS, _ROWS)],
                         osems[b])

    for b in range(_NBUF):
        p0_fire_in(s + 16 * b, b)

    def p0_body(kk, _):
        for b in range(_NBUF):
            k = kk * _NBUF + b
            j = s + 16 * k
            actn = (k + _NBUF < _CP_SLOTS - 1) | ((k + _NBUF == _CP_SLOTS - 1)
                                                  & (s < 4))
            p0_wait_in(b)
            p0_fire_out(j, b)

            @pl.when(actn)
            def _():
                p0_wait_out(b)
                p0_fire_in(s + 16 * (k + _NBUF), b)
        return 0

    lax.fori_loop(0, (_CP_SLOTS - 2) // _NBUF, p0_body, 0)
    p0_wait_in(0)
    p0_fire_out(s + 16 * (_CP_SLOTS - 2), 0)

    @pl.when(s < 4)
    def _():
        p0_wait_in(1)
        p0_fire_out(s + 16 * (_CP_SLOTS - 1), 1)

    for b in range(_NBUF):
        p0_wait_out(b)

    @pl.when(s < _NEW_CH)
    def _():
        pltpu.sync_copy(new_hbm.at[pl.ds(s * _ROWS, _ROWS)], bufs[0])
        pltpu.sync_copy(bufs[0],
                        table_s.at[c, pl.ds(_NUM_EMB + s * _ROWS, _ROWS)])

    @pl.when(s == _NEW_CH)
    def _():
        pltpu.sync_copy(new_hbm.at[pl.ds(_NEW_CH * _ROWS, _NEW_TAIL)],
                        bufs[0].at[pl.ds(0, _NEW_TAIL)])
        pltpu.sync_copy(
            bufs[0].at[pl.ds(0, _NEW_TAIL)],
            table_s.at[c, pl.ds(_NUM_EMB + _NEW_CH * _ROWS, _NEW_TAIL)])

    plsc.subcore_barrier()

    def wait_gather(b):
        pltpu.make_async_copy(
            table_s.at[c, pl.ds(0, _ROWS)], bufs[b], gsems[b]).wait()

    def wait_scatter(b):
        pltpu.make_async_copy(
            bufs[b], table_s.at[c, pl.ds(0, _ROWS)], osems[b]).wait()

    def fire_gather(g, b):
        for kb in range(_KB):
            i = g * _KB + kb
            pltpu.async_copy(
                table_s.at[c].at[idx_v.at[pl.ds(i * _HPAD, _HIST)]],
                bufs[b].at[pl.ds(kb * _HIST, _HIST)], gsems[b])

    def fire_scatter(g, b):
        for kb in range(_KB):
            pltpu.async_copy(bufs[b].at[pl.ds(kb * _HIST, _HIST)],
                             out_hbm.at[bt0 + g * _KB + kb], osems[b])

    for b in range(_NBUF):
        fire_gather(b, b)

    def body(gg, _):
        for b in range(_NBUF):
            g = gg * _NBUF + b
            wait_gather(b)
            fire_scatter(g, b)

            @pl.when(g + _NBUF < _GRP)
            def _():
                wait_scatter(b)
                fire_gather(g + _NBUF, b)
        return 0

    lax.fori_loop(0, _GRP // _NBUF, body, 0)
    for g, b in ((_GRP - 2, 0), (_GRP - 1, 1)):
        wait_gather(b)
        fire_scatter(g, b)
    for b in range(_NBUF):
        wait_scatter(b)


def kernel(x, embedding, new_embedding):
    xi = x.astype(jnp.int32)
    xpad = jnp.pad(xi, ((0, 0), (0, _HPAD - _HIST))).reshape(-1)

    mesh = plsc.VectorSubcoreMesh(core_axis_name="c", subcore_axis_name="s")
    run = pl.kernel(
        _gather_body,
        out_type=jax.ShapeDtypeStruct((_BATCH, _HIST, _FEATURES),
                                      jnp.float32),
        mesh=mesh,
        scratch_types=[
            pltpu.HBM((_NC, _TABLE, _FEATURES), jnp.float32),
            pltpu.VMEM((_BT_PER_W * _HPAD,), jnp.int32),
            tuple(pltpu.VMEM((_ROWS, _FEATURES), jnp.float32)
                  for _ in range(_NBUF)),
            tuple(pltpu.SemaphoreType.DMA for _ in range(_NBUF)),
            tuple(pltpu.SemaphoreType.DMA for _ in range(_NBUF)),
        ],
    )
    return run(xpad, embedding, new_embedding)
